# baseline (speedup 1.0000x reference)
import functools
import jax
import jax.numpy as jnp
from jax.experimental import pallas as pl
from jax.experimental.pallas import tpu as pltpu

NSAMPLE = 16
HID = 64


def _leaky(x):
    return jnp.where(x >= 0, x, 0.1 * x)


def _knn(queries, refs, k):
    qn = jnp.sum(queries * queries, axis=-1)
    rn = jnp.sum(refs * refs, axis=-1)
    d = qn[:, None] + rn[None, :] - 2.0 * (queries @ refs.T)
    _, idx = jax.lax.top_k(-d, k)
    return idx


def _gn_nk(y, gamma, beta, groups, eps=1e-5):
    N, K, C = y.shape
    g = y.reshape(N, K, groups, C // groups)
    m = jnp.mean(g, axis=(0, 1, 3), keepdims=True)
    v = jnp.var(g, axis=(0, 1, 3), keepdims=True)
    g = (g - m) / jnp.sqrt(v + eps)
    return g.reshape(N, K, C) * gamma[None, None, :] + beta[None, None, :]


def _wsum_kernel(w_ref, v_ref, o_ref):
    o_ref[...] = jnp.sum(w_ref[...] * v_ref[...], axis=1)


def _batch_one(xyz0, xyz1, state, corr0, feat0, feat1, flow0, W):
    k = NSAMPLE
    P0 = xyz0.T
    P1 = xyz1.T
    N = P0.shape[0]

    idx0 = _knn(P0, P0, k)
    idx1 = _knn(P1, P0, k)

    f0t = flow0.T
    gf = f0t[idx0]
    gx = P0[idx0] - P0[:, None, :]
    in1 = jnp.concatenate([gf, gx], axis=-1)
    y1 = in1 @ W['flow_w1'].T + W['flow_b1']
    h1 = _leaky(_gn_nk(y1, W['flow_g1'], W['flow_be1'], 4))
    y2 = h1 @ W['flow_w2'].T + W['flow_b2']
    h2 = _leaky(_gn_nk(y2, W['flow_g2'], W['flow_be2'], 4))
    flow_feat0 = jnp.max(h2, axis=1)

    feat = jnp.concatenate([corr0.T, flow_feat0], axis=-1)
    st = state.T
    hs = jnp.concatenate([feat, st], axis=-1)

    def table(Wc, x_parts):
        out = 0.0
        ofs = 0
        for xp in x_parts:
            out = out + xp @ Wc[:, ofs:ofs + xp.shape[1]].T
            ofs += xp.shape[1]
        out = out + P0 @ Wc[:, ofs:ofs + 3].T
        return out

    pxz = P0 @ W['convz_w'][:, 192:195].T
    pxr = P0 @ W['convr_w'][:, 192:195].T
    pxq = P0 @ W['convq_w'][:, 192:195].T
    Az = table(W['convz_w'], [hs])
    Ar = table(W['convr_w'], [hs])
    z = jax.nn.sigmoid(jnp.max(Az[idx0], axis=1) - pxz + W['convz_b'])
    r = jax.nn.sigmoid(jnp.max(Ar[idx0], axis=1) - pxr + W['convr_b'])
    Sq = table(W['convq_w'], [feat, r * st])
    q = jnp.tanh(jnp.max(Sq[idx0], axis=1) - pxq + W['convq_b'])
    next_state = (1.0 - z) * st + z * q

    G = feat0.T @ W['interp_w1'][:, 0:64].T + P0 @ W['interp_w1'][:, 128:131].T
    Q = feat1.T @ W['interp_w1'][:, 64:128].T - P1 @ W['interp_w1'][:, 128:131].T \
        + W['interp_b1']
    h = _leaky(G[idx1] + Q[:, None, :])
    y = h @ W['interp_w2'].T + W['interp_b2']
    wgt = jax.nn.softmax(y, axis=1)
    V = next_state[idx1]
    blk = 512
    out = pl.pallas_call(
        _wsum_kernel,
        grid=(N // blk,),
        in_specs=[pl.BlockSpec((blk, 16, 64), lambda i: (i, 0, 0)),
                  pl.BlockSpec((blk, 16, 64), lambda i: (i, 0, 0))],
        out_specs=pl.BlockSpec((blk, 64), lambda i: (i, 0)),
        out_shape=jax.ShapeDtypeStruct((N, 64), jnp.float32),
    )(wgt, V)
    return out.T


def kernel(xyz0, xyz1, state, corr0, feat0, feat1, flow0,
           flow_w1, flow_b1, flow_g1, flow_be1,
           flow_w2, flow_b2, flow_g2, flow_be2,
           convz_w, convz_b, convr_w, convr_b, convq_w, convq_b,
           interp_w1, interp_b1, interp_w2, interp_b2):
    W = dict(flow_w1=flow_w1, flow_b1=flow_b1, flow_g1=flow_g1,
             flow_be1=flow_be1, flow_w2=flow_w2, flow_b2=flow_b2,
             flow_g2=flow_g2, flow_be2=flow_be2,
             convz_w=convz_w, convz_b=convz_b, convr_w=convr_w,
             convr_b=convr_b, convq_w=convq_w, convq_b=convq_b,
             interp_w1=interp_w1, interp_b1=interp_b1,
             interp_w2=interp_w2, interp_b2=interp_b2)
    f = jax.vmap(lambda a, b, c, d, e, g, h: _batch_one(a, b, c, d, e, g, h, W),
                 in_axes=0)
    return f(xyz0, xyz1, state, corr0, feat0, feat1, flow0)


# speedup vs baseline: 2.2794x; 2.2794x over previous
import functools
import jax
import jax.numpy as jnp
from jax.experimental import pallas as pl
from jax.experimental.pallas import tpu as pltpu

NSAMPLE = 16
HID = 64


def _leaky(x):
    return jnp.where(x >= 0, x, 0.1 * x)


QBLK = 256


def _knn_body(q_ref, rt_ref, o_ref):
    q = q_ref[...]
    rt = rt_ref[...]
    n = rt.shape[1]
    qn = jnp.sum(q * q, axis=1, keepdims=True)
    rn = jnp.sum(rt * rt, axis=0, keepdims=True)
    d = qn + rn - 2.0 * jnp.dot(q, rt, preferred_element_type=jnp.float32)
    iota = jax.lax.broadcasted_iota(jnp.int32, d.shape, 1)
    cols = []
    for _ in range(NSAMPLE):
        m = jnp.min(d, axis=1, keepdims=True)
        mi = jnp.where(d <= m, iota, jnp.int32(n))
        j = jnp.min(mi, axis=1, keepdims=True)
        cols.append(j)
        d = jnp.where(iota == j, jnp.float32(jnp.inf), d)
    o_ref[...] = jnp.concatenate(cols, axis=1)


def _knn(queries, refs, k):
    del k
    N = queries.shape[0]
    return pl.pallas_call(
        _knn_body,
        grid=(N // QBLK,),
        in_specs=[pl.BlockSpec((QBLK, 3), lambda i: (i, 0)),
                  pl.BlockSpec((3, refs.shape[0]), lambda i: (0, 0))],
        out_specs=pl.BlockSpec((QBLK, NSAMPLE), lambda i: (i, 0)),
        out_shape=jax.ShapeDtypeStruct((N, NSAMPLE), jnp.int32),
    )(queries, refs.T)


def _gn_nk(y, gamma, beta, groups, eps=1e-5):
    N, K, C = y.shape
    g = y.reshape(N, K, groups, C // groups)
    m = jnp.mean(g, axis=(0, 1, 3), keepdims=True)
    v = jnp.var(g, axis=(0, 1, 3), keepdims=True)
    g = (g - m) / jnp.sqrt(v + eps)
    return g.reshape(N, K, C) * gamma[None, None, :] + beta[None, None, :]


def _wsum_kernel(w_ref, v_ref, o_ref):
    o_ref[...] = jnp.sum(w_ref[...] * v_ref[...], axis=1)


def _batch_one(xyz0, xyz1, state, corr0, feat0, feat1, flow0, W):
    k = NSAMPLE
    P0 = xyz0.T
    P1 = xyz1.T
    N = P0.shape[0]

    idx0 = _knn(P0, P0, k)
    idx1 = _knn(P1, P0, k)

    f0t = flow0.T
    gf = f0t[idx0]
    gx = P0[idx0] - P0[:, None, :]
    in1 = jnp.concatenate([gf, gx], axis=-1)
    y1 = in1 @ W['flow_w1'].T + W['flow_b1']
    h1 = _leaky(_gn_nk(y1, W['flow_g1'], W['flow_be1'], 4))
    y2 = h1 @ W['flow_w2'].T + W['flow_b2']
    h2 = _leaky(_gn_nk(y2, W['flow_g2'], W['flow_be2'], 4))
    flow_feat0 = jnp.max(h2, axis=1)

    feat = jnp.concatenate([corr0.T, flow_feat0], axis=-1)
    st = state.T
    hs = jnp.concatenate([feat, st], axis=-1)

    def table(Wc, x_parts):
        out = 0.0
        ofs = 0
        for xp in x_parts:
            out = out + xp @ Wc[:, ofs:ofs + xp.shape[1]].T
            ofs += xp.shape[1]
        out = out + P0 @ Wc[:, ofs:ofs + 3].T
        return out

    pxz = P0 @ W['convz_w'][:, 192:195].T
    pxr = P0 @ W['convr_w'][:, 192:195].T
    pxq = P0 @ W['convq_w'][:, 192:195].T
    Az = table(W['convz_w'], [hs])
    Ar = table(W['convr_w'], [hs])
    z = jax.nn.sigmoid(jnp.max(Az[idx0], axis=1) - pxz + W['convz_b'])
    r = jax.nn.sigmoid(jnp.max(Ar[idx0], axis=1) - pxr + W['convr_b'])
    Sq = table(W['convq_w'], [feat, r * st])
    q = jnp.tanh(jnp.max(Sq[idx0], axis=1) - pxq + W['convq_b'])
    next_state = (1.0 - z) * st + z * q

    G = feat0.T @ W['interp_w1'][:, 0:64].T + P0 @ W['interp_w1'][:, 128:131].T
    Q = feat1.T @ W['interp_w1'][:, 64:128].T - P1 @ W['interp_w1'][:, 128:131].T \
        + W['interp_b1']
    h = _leaky(G[idx1] + Q[:, None, :])
    y = h @ W['interp_w2'].T + W['interp_b2']
    wgt = jax.nn.softmax(y, axis=1)
    V = next_state[idx1]
    blk = 512
    out = pl.pallas_call(
        _wsum_kernel,
        grid=(N // blk,),
        in_specs=[pl.BlockSpec((blk, 16, 64), lambda i: (i, 0, 0)),
                  pl.BlockSpec((blk, 16, 64), lambda i: (i, 0, 0))],
        out_specs=pl.BlockSpec((blk, 64), lambda i: (i, 0)),
        out_shape=jax.ShapeDtypeStruct((N, 64), jnp.float32),
    )(wgt, V)
    return out.T


def kernel(xyz0, xyz1, state, corr0, feat0, feat1, flow0,
           flow_w1, flow_b1, flow_g1, flow_be1,
           flow_w2, flow_b2, flow_g2, flow_be2,
           convz_w, convz_b, convr_w, convr_b, convq_w, convq_b,
           interp_w1, interp_b1, interp_w2, interp_b2):
    W = dict(flow_w1=flow_w1, flow_b1=flow_b1, flow_g1=flow_g1,
             flow_be1=flow_be1, flow_w2=flow_w2, flow_b2=flow_b2,
             flow_g2=flow_g2, flow_be2=flow_be2,
             convz_w=convz_w, convz_b=convz_b, convr_w=convr_w,
             convr_b=convr_b, convq_w=convq_w, convq_b=convq_b,
             interp_w1=interp_w1, interp_b1=interp_b1,
             interp_w2=interp_w2, interp_b2=interp_b2)
    f = jax.vmap(lambda a, b, c, d, e, g, h: _batch_one(a, b, c, d, e, g, h, W),
                 in_axes=0)
    return f(xyz0, xyz1, state, corr0, feat0, feat1, flow0)


# speedup vs baseline: 16.0017x; 7.0203x over previous
import functools
import jax
import jax.numpy as jnp
from jax import lax
from jax.experimental import pallas as pl
from jax.experimental.pallas import tpu as pltpu
from jax.experimental.pallas import tpu_sc as plsc

NSAMPLE = 16
HID = 64
N = 4096
B = 2
QBLK = 256
EPS = 1e-5


def _leaky(x):
    return jnp.where(x >= 0, x, 0.1 * x)


def _knn_body(q_ref, rt_ref, o_ref):
    q = q_ref[...]
    rt = rt_ref[...]
    n = rt.shape[1]
    qn = jnp.sum(q * q, axis=1, keepdims=True)
    rn = jnp.sum(rt * rt, axis=0, keepdims=True)
    d = qn + rn - 2.0 * jnp.dot(q, rt, preferred_element_type=jnp.float32)
    iota = lax.broadcasted_iota(jnp.int32, d.shape, 1)
    cols = []
    for _ in range(NSAMPLE):
        m = jnp.min(d, axis=1, keepdims=True)
        mi = jnp.where(d <= m, iota, jnp.int32(n))
        j = jnp.min(mi, axis=1, keepdims=True)
        cols.append(j)
        d = jnp.where(iota == j, jnp.float32(jnp.inf), d)
    o_ref[...] = jnp.concatenate(cols, axis=1)


def _knn(queries, refs):
    n = queries.shape[0]
    return pl.pallas_call(
        _knn_body,
        grid=(n // QBLK,),
        in_specs=[pl.BlockSpec((QBLK, 3), lambda i: (i, 0)),
                  pl.BlockSpec((3, refs.shape[0]), lambda i: (0, 0))],
        out_specs=pl.BlockSpec((QBLK, NSAMPLE), lambda i: (i, 0)),
        out_shape=jax.ShapeDtypeStruct((n, NSAMPLE), jnp.int32),
    )(queries, refs.T)


def _sc_gather(table, idx):
    M = idx.shape[0]
    D = table.shape[1]
    NW = 32
    per_w = M // NW
    ch = min(per_w, 512)
    mesh = plsc.VectorSubcoreMesh(core_axis_name="c", subcore_axis_name="s")

    @functools.partial(
        pl.kernel, mesh=mesh,
        out_type=jax.ShapeDtypeStruct((M, D), jnp.float32),
        scratch_types=[
            pltpu.VMEM((ch,), jnp.int32),
            pltpu.VMEM((ch, D), jnp.float32),
            pltpu.SemaphoreType.DMA,
        ],
    )
    def k(table_hbm, idx_hbm, out_hbm, idx_v, rows_v, sem):
        wid = lax.axis_index("s") * 2 + lax.axis_index("c")
        base = wid * per_w

        @pl.loop(0, per_w, step=ch)
        def _(off):
            pltpu.sync_copy(idx_hbm.at[pl.ds(base + off, ch)], idx_v)
            pltpu.async_copy(table_hbm.at[idx_v], rows_v, sem).wait()
            pltpu.sync_copy(rows_v, out_hbm.at[pl.ds(base + off, ch)])

    return k(table, idx)


FBLK = 1024
FNB = N // FBLK
_FCNT = float(N * NSAMPLE * 16)


def _y1_block(r1, px, b1):
    pb = jnp.broadcast_to(px[:, None, :], (FBLK, NSAMPLE, 64)).reshape(
        FBLK * NSAMPLE, 64)
    return r1[:, 0:64] - pb + b1


def _gstats(y):
    parts = []
    for g in range(4):
        blkg = y[:, 16 * g:16 * (g + 1)]
        parts.append(jnp.sum(blkg).reshape(1, 1, 1))
    for g in range(4):
        blkg = y[:, 16 * g:16 * (g + 1)]
        parts.append(jnp.sum(blkg * blkg).reshape(1, 1, 1))
    return jnp.concatenate(parts, axis=2).reshape(1, 1, 1, 8)


def _gn_apply(y, stats, gamma, beta):
    s = jnp.sum(stats.reshape(1, FNB, 8), axis=1)
    outs = []
    for g in range(4):
        m = s[0, g] / _FCNT
        v = s[0, 4 + g] / _FCNT - m * m
        blkg = y[:, 16 * g:16 * (g + 1)]
        outs.append((blkg - m) * lax.rsqrt(v + EPS))
    yn = jnp.concatenate(outs, axis=1) * gamma + beta
    return _leaky(yn)


def _flow_s1_body(r1_ref, px_ref, b1_ref, st_ref):
    st_ref[...] = _gstats(_y1_block(r1_ref[...], px_ref[...], b1_ref[...]))


def _flow_s2_body(r1_ref, px_ref, b1_ref, st1_ref, g1_ref, be1_ref,
                  w2_ref, b2_ref, y2_ref, st2_ref):
    y1 = _y1_block(r1_ref[...], px_ref[...], b1_ref[...])
    h = _gn_apply(y1, st1_ref[...], g1_ref[...], be1_ref[...])
    y2 = jnp.dot(h, w2_ref[...].T, preferred_element_type=jnp.float32) \
        + b2_ref[...]
    y2_ref[...] = y2
    st2_ref[...] = _gstats(y2)


def _flow_s3_body(y2_ref, st2_ref, g2_ref, be2_ref, o_ref):
    h2 = _gn_apply(y2_ref[...], st2_ref[...], g2_ref[...], be2_ref[...])
    o_ref[...] = jnp.max(h2.reshape(FBLK, NSAMPLE, 64), axis=1)


def _flow_stage(r1, px1, b1, g1, be1, w2, b2, g2, be2):
    grid = (B, FNB)
    rblk = pl.BlockSpec((FBLK * NSAMPLE, 128), lambda b, i: (b * FNB + i, 0))
    pblk = pl.BlockSpec((FBLK, 64), lambda b, i: (b * FNB + i, 0))
    vec = pl.BlockSpec((64,), lambda b, i: (0,))
    st_out = pl.BlockSpec((1, 1, 1, 8), lambda b, i: (b, i, 0, 0))
    st_in = pl.BlockSpec((1, FNB, 1, 8), lambda b, i: (b, 0, 0, 0))
    st_shape = jax.ShapeDtypeStruct((B, FNB, 1, 8), jnp.float32)

    st1 = pl.pallas_call(
        _flow_s1_body, grid=grid,
        in_specs=[rblk, pblk, vec],
        out_specs=st_out, out_shape=st_shape,
    )(r1, px1, b1)

    y2, st2 = pl.pallas_call(
        _flow_s2_body, grid=grid,
        in_specs=[rblk, pblk, vec, st_in, vec, vec,
                  pl.BlockSpec((64, 64), lambda b, i: (0, 0)), vec],
        out_specs=[pl.BlockSpec((FBLK * NSAMPLE, 64),
                                lambda b, i: (b * FNB + i, 0)), st_out],
        out_shape=[jax.ShapeDtypeStruct((B * N * NSAMPLE, 64), jnp.float32),
                   st_shape],
    )(r1, px1, b1, st1, g1, be1, w2, b2)

    return pl.pallas_call(
        _flow_s3_body, grid=grid,
        in_specs=[pl.BlockSpec((FBLK * NSAMPLE, 64),
                               lambda b, i: (b * FNB + i, 0)),
                  st_in, vec, vec],
        out_specs=pblk,
        out_shape=jax.ShapeDtypeStruct((B * N, 64), jnp.float32),
    )(y2, st2, g2, be2)


def _a1_body(fl_ref, p0_ref, w1_ref, a1_ref, px_ref):
    w1 = w1_ref[...]
    a1 = (jnp.dot(fl_ref[...], w1[:, 0:3].T, preferred_element_type=jnp.float32)
          + jnp.dot(p0_ref[...], w1[:, 3:6].T, preferred_element_type=jnp.float32))
    a1_ref[...] = jnp.concatenate(
        [a1, jnp.zeros((a1.shape[0], 64), jnp.float32)], axis=1)
    px_ref[...] = jnp.dot(p0_ref[...], w1[:, 3:6].T,
                          preferred_element_type=jnp.float32)


def _a1_pre(fl0, p0s, w1):
    blk = lambda d: pl.BlockSpec((N, d), lambda b: (b, 0))
    return pl.pallas_call(
        _a1_body,
        grid=(B,),
        in_specs=[blk(3), blk(3), pl.BlockSpec((64, 6), lambda b: (0, 0))],
        out_specs=[blk(128), blk(64)],
        out_shape=[jax.ShapeDtypeStruct((B * N, 128), jnp.float32),
                   jax.ShapeDtypeStruct((B * N, 64), jnp.float32)],
    )(fl0, p0s, w1)


def _gru_pre_body(c_ref, f_ref, s_ref, p_ref, wz_ref, wr_ref, azr_ref, px_ref):
    hs = jnp.concatenate([c_ref[...], f_ref[...], s_ref[...]], axis=1)
    p0 = p_ref[...]
    wz = wz_ref[...]
    wr = wr_ref[...]
    az = (jnp.dot(hs, wz[:, 0:192].T, preferred_element_type=jnp.float32)
          + jnp.dot(p0, wz[:, 192:195].T, preferred_element_type=jnp.float32))
    ar = (jnp.dot(hs, wr[:, 0:192].T, preferred_element_type=jnp.float32)
          + jnp.dot(p0, wr[:, 192:195].T, preferred_element_type=jnp.float32))
    azr_ref[...] = jnp.concatenate([az, ar], axis=1)
    pxz = jnp.dot(p0, wz[:, 192:195].T, preferred_element_type=jnp.float32)
    pxr = jnp.dot(p0, wr[:, 192:195].T, preferred_element_type=jnp.float32)
    px_ref[...] = jnp.concatenate([pxz, pxr], axis=1)


def _gru_pre(c, f, s, p, wz, wr):
    blk = lambda d: pl.BlockSpec((N, d), lambda b: (b, 0))
    return pl.pallas_call(
        _gru_pre_body,
        grid=(B,),
        in_specs=[blk(64), blk(64), blk(64), blk(3),
                  pl.BlockSpec((64, 195), lambda b: (0, 0)),
                  pl.BlockSpec((64, 195), lambda b: (0, 0))],
        out_specs=[blk(128), blk(128)],
        out_shape=[jax.ShapeDtypeStruct((B * N, 128), jnp.float32),
                   jax.ShapeDtypeStruct((B * N, 128), jnp.float32)],
    )(c, f, s, p, wz, wr)


ZBLK = 1024


def _gru_mid_body(g_ref, px_ref, c_ref, f_ref, s_ref, p_ref,
                  wq_ref, bz_ref, br_ref, sq_ref, z_ref, pxq_ref):
    mzr = jnp.max(g_ref[...].reshape(ZBLK, NSAMPLE, 128), axis=1)
    px = px_ref[...]
    z = jax.nn.sigmoid(mzr[:, 0:64] - px[:, 0:64] + bz_ref[...])
    r = jax.nn.sigmoid(mzr[:, 64:128] - px[:, 64:128] + br_ref[...])
    st = s_ref[...]
    rs = r * st
    feat = jnp.concatenate([c_ref[...], f_ref[...]], axis=1)
    wq = wq_ref[...]
    p0 = p_ref[...]
    sq = (jnp.dot(feat, wq[:, 0:128].T, preferred_element_type=jnp.float32)
          + jnp.dot(rs, wq[:, 128:192].T, preferred_element_type=jnp.float32)
          + jnp.dot(p0, wq[:, 192:195].T, preferred_element_type=jnp.float32))
    sq_ref[...] = jnp.concatenate(
        [sq, jnp.zeros((sq.shape[0], 64), jnp.float32)], axis=1)
    z_ref[...] = z
    pxq_ref[...] = jnp.dot(p0, wq[:, 192:195].T, preferred_element_type=jnp.float32)


def _gru_mid(gzr, px, c, f, s, p, wq, bz, br):
    nb = (B * N) // ZBLK
    blk = lambda d: pl.BlockSpec((ZBLK, d), lambda g: (g, 0))
    vec = pl.BlockSpec((64,), lambda g: (0,))
    return pl.pallas_call(
        _gru_mid_body,
        grid=(nb,),
        in_specs=[pl.BlockSpec((ZBLK * NSAMPLE, 128), lambda g: (g, 0)),
                  blk(128), blk(64), blk(64), blk(64), blk(3),
                  pl.BlockSpec((64, 195), lambda g: (0, 0)), vec, vec],
        out_specs=[blk(128), blk(64), blk(64)],
        out_shape=[jax.ShapeDtypeStruct((B * N, 128), jnp.float32),
                   jax.ShapeDtypeStruct((B * N, 64), jnp.float32),
                   jax.ShapeDtypeStruct((B * N, 64), jnp.float32)],
    )(gzr, px, c, f, s, p, wq, bz, br)


def _gru_fin_body(g_ref, pxq_ref, z_ref, s_ref, p0_ref, p1_ref,
                  f0_ref, f1_ref, bq_ref, wa_ref, ba_ref, gv_ref, qq_ref):
    mq = jnp.max(g_ref[...][:, 0:64].reshape(ZBLK, NSAMPLE, 64), axis=1)
    q = jnp.tanh(mq - pxq_ref[...] + bq_ref[...])
    z = z_ref[...]
    ns = (1.0 - z) * s_ref[...] + z * q
    wa = wa_ref[...]
    g_t = (jnp.dot(f0_ref[...], wa[:, 0:64].T, preferred_element_type=jnp.float32)
           + jnp.dot(p0_ref[...], wa[:, 128:131].T, preferred_element_type=jnp.float32))
    qq = (jnp.dot(f1_ref[...], wa[:, 64:128].T, preferred_element_type=jnp.float32)
          - jnp.dot(p1_ref[...], wa[:, 128:131].T, preferred_element_type=jnp.float32)
          + ba_ref[...])
    gv_ref[...] = jnp.concatenate([g_t, ns], axis=1)
    qq_ref[...] = qq


def _gru_fin(gq, pxq, z, s, p0, p1, f0, f1, bq, wa, ba):
    nb = (B * N) // ZBLK
    blk = lambda d: pl.BlockSpec((ZBLK, d), lambda g: (g, 0))
    vec = pl.BlockSpec((64,), lambda g: (0,))
    return pl.pallas_call(
        _gru_fin_body,
        grid=(nb,),
        in_specs=[pl.BlockSpec((ZBLK * NSAMPLE, 128), lambda g: (g, 0)),
                  blk(64), blk(64), blk(64), blk(3), blk(3), blk(64), blk(64),
                  vec, pl.BlockSpec((64, 131), lambda g: (0, 0)), vec],
        out_specs=[blk(128), blk(64)],
        out_shape=[jax.ShapeDtypeStruct((B * N, 128), jnp.float32),
                   jax.ShapeDtypeStruct((B * N, 64), jnp.float32)],
    )(gq, pxq, z, s, p0, p1, f0, f1, bq, wa, ba)


ABLK = 512


def _align_body(r2_ref, qq_ref, w2_ref, b2_ref, o_ref):
    r2 = r2_ref[...]
    qq = qq_ref[...]
    qb = jnp.broadcast_to(qq[:, None, :], (ABLK, NSAMPLE, 64)).reshape(
        ABLK * NSAMPLE, 64)
    h = _leaky(r2[:, 0:64] + qb)
    y = jnp.dot(h, w2_ref[...].T, preferred_element_type=jnp.float32) + b2_ref[...]
    y3 = y.reshape(ABLK, NSAMPLE, 64)
    m = jnp.max(y3, axis=1, keepdims=True)
    e = jnp.exp(y3 - m)
    w = e / jnp.sum(e, axis=1, keepdims=True)
    v3 = r2[:, 64:128].reshape(ABLK, NSAMPLE, 64)
    o_ref[...] = jnp.sum(w * v3, axis=1)


def _align(r2, qq, w2, b2):
    nb = (B * N) // ABLK
    return pl.pallas_call(
        _align_body,
        grid=(nb,),
        in_specs=[pl.BlockSpec((ABLK * NSAMPLE, 128), lambda g: (g, 0)),
                  pl.BlockSpec((ABLK, 64), lambda g: (g, 0)),
                  pl.BlockSpec((64, 64), lambda g: (0, 0)),
                  pl.BlockSpec((64,), lambda g: (0,))],
        out_specs=pl.BlockSpec((ABLK, 64), lambda g: (g, 0)),
        out_shape=jax.ShapeDtypeStruct((B * N, 64), jnp.float32),
    )(r2, qq, w2, b2)


def kernel(xyz0, xyz1, state, corr0, feat0, feat1, flow0,
           flow_w1, flow_b1, flow_g1, flow_be1,
           flow_w2, flow_b2, flow_g2, flow_be2,
           convz_w, convz_b, convr_w, convr_b, convq_w, convq_b,
           interp_w1, interp_b1, interp_w2, interp_b2):
    t = lambda x: x.transpose(0, 2, 1).reshape(B * N, -1)
    p0s = t(xyz0)
    p1s = t(xyz1)
    sts = t(state)
    c0s = t(corr0)
    f0s = t(feat0)
    f1s = t(feat1)
    fl0 = t(flow0)

    idx0 = jax.vmap(_knn)(p0s.reshape(B, N, 3), p0s.reshape(B, N, 3))
    idx1 = jax.vmap(_knn)(p1s.reshape(B, N, 3), p0s.reshape(B, N, 3))
    offs = (jnp.arange(B, dtype=jnp.int32) * N)[:, None, None]
    idxg0 = (idx0 + offs).reshape(-1)
    idxg1 = (idx1 + offs).reshape(-1)

    a1, px1 = _a1_pre(fl0, p0s, flow_w1)
    r1 = _sc_gather(a1, idxg0)
    ff0 = _flow_stage(r1, px1, flow_b1, flow_g1, flow_be1,
                      flow_w2, flow_b2, flow_g2, flow_be2)

    azr, px = _gru_pre(c0s, ff0, sts, p0s, convz_w, convr_w)
    gzr = _sc_gather(azr, idxg0)
    sq, z, pxq = _gru_mid(gzr, px, c0s, ff0, sts, p0s, convq_w,
                          convz_b, convr_b)
    gq = _sc_gather(sq, idxg0)
    gv, qq = _gru_fin(gq, pxq, z, sts, p0s, p1s, f0s, f1s,
                      convq_b, interp_w1, interp_b1)

    r2 = _sc_gather(gv, idxg1)
    out = _align(r2, qq, interp_w2, interp_b2)
    return out.reshape(B, N, 64).transpose(0, 2, 1)


# speedup vs baseline: 16.0025x; 1.0000x over previous
import functools
import jax
import jax.numpy as jnp
from jax import lax
from jax.experimental import pallas as pl
from jax.experimental.pallas import tpu as pltpu
from jax.experimental.pallas import tpu_sc as plsc

NSAMPLE = 16
HID = 64
N = 4096
B = 2
QBLK = 256
EPS = 1e-5


def _leaky(x):
    return jnp.where(x >= 0, x, 0.1 * x)


def _knn_body(q_ref, rt_ref, o_ref):
    q = q_ref[...]
    rt = rt_ref[...]
    n = rt.shape[1]
    qn = jnp.sum(q * q, axis=1, keepdims=True)
    rn = jnp.sum(rt * rt, axis=0, keepdims=True)
    d = qn + rn - 2.0 * jnp.dot(q, rt, preferred_element_type=jnp.float32)
    iota = lax.broadcasted_iota(jnp.int32, d.shape, 1)
    cols = []
    for _ in range(NSAMPLE):
        m = jnp.min(d, axis=1, keepdims=True)
        mi = jnp.where(d <= m, iota, jnp.int32(n))
        j = jnp.min(mi, axis=1, keepdims=True)
        cols.append(j)
        d = jnp.where(iota == j, jnp.float32(jnp.inf), d)
    o_ref[...] = jnp.concatenate(cols, axis=1)


def _knn(queries, refs):
    n = queries.shape[0]
    return pl.pallas_call(
        _knn_body,
        grid=(n // QBLK,),
        in_specs=[pl.BlockSpec((QBLK, 3), lambda i: (i, 0)),
                  pl.BlockSpec((3, refs.shape[0]), lambda i: (0, 0))],
        out_specs=pl.BlockSpec((QBLK, NSAMPLE), lambda i: (i, 0)),
        out_shape=jax.ShapeDtypeStruct((n, NSAMPLE), jnp.int32),
        compiler_params=pltpu.CompilerParams(
            dimension_semantics=("parallel",)),
    )(queries, refs.T)


def _sc_gather(table, idx):
    M = idx.shape[0]
    D = table.shape[1]
    NW = 32
    per_w = M // NW
    ch = min(per_w, 512)
    mesh = plsc.VectorSubcoreMesh(core_axis_name="c", subcore_axis_name="s")

    @functools.partial(
        pl.kernel, mesh=mesh,
        out_type=jax.ShapeDtypeStruct((M, D), jnp.float32),
        scratch_types=[
            pltpu.VMEM((ch,), jnp.int32),
            pltpu.VMEM((ch, D), jnp.float32),
            pltpu.SemaphoreType.DMA,
        ],
    )
    def k(table_hbm, idx_hbm, out_hbm, idx_v, rows_v, sem):
        wid = lax.axis_index("s") * 2 + lax.axis_index("c")
        base = wid * per_w

        @pl.loop(0, per_w, step=ch)
        def _(off):
            pltpu.sync_copy(idx_hbm.at[pl.ds(base + off, ch)], idx_v)
            pltpu.async_copy(table_hbm.at[idx_v], rows_v, sem).wait()
            pltpu.sync_copy(rows_v, out_hbm.at[pl.ds(base + off, ch)])

    return k(table, idx)


FBLK = 1024
FNB = N // FBLK
_FCNT = float(N * NSAMPLE * 16)


def _y1_block(r1, px, b1):
    pb = jnp.broadcast_to(px[:, None, :], (FBLK, NSAMPLE, 64)).reshape(
        FBLK * NSAMPLE, 64)
    return r1[:, 0:64] - pb + b1


def _gstats(y):
    parts = []
    for g in range(4):
        blkg = y[:, 16 * g:16 * (g + 1)]
        parts.append(jnp.sum(blkg).reshape(1, 1, 1))
    for g in range(4):
        blkg = y[:, 16 * g:16 * (g + 1)]
        parts.append(jnp.sum(blkg * blkg).reshape(1, 1, 1))
    return jnp.concatenate(parts, axis=2).reshape(1, 1, 1, 8)


def _gn_apply(y, stats, gamma, beta):
    s = jnp.sum(stats.reshape(1, FNB, 8), axis=1)
    outs = []
    for g in range(4):
        m = s[0, g] / _FCNT
        v = s[0, 4 + g] / _FCNT - m * m
        blkg = y[:, 16 * g:16 * (g + 1)]
        outs.append((blkg - m) * lax.rsqrt(v + EPS))
    yn = jnp.concatenate(outs, axis=1) * gamma + beta
    return _leaky(yn)


def _flow_s1_body(r1_ref, px_ref, b1_ref, st_ref):
    st_ref[...] = _gstats(_y1_block(r1_ref[...], px_ref[...], b1_ref[...]))


def _flow_s2_body(r1_ref, px_ref, b1_ref, st1_ref, g1_ref, be1_ref,
                  w2_ref, b2_ref, y2_ref, st2_ref):
    y1 = _y1_block(r1_ref[...], px_ref[...], b1_ref[...])
    h = _gn_apply(y1, st1_ref[...], g1_ref[...], be1_ref[...])
    y2 = jnp.dot(h, w2_ref[...].T, preferred_element_type=jnp.float32) \
        + b2_ref[...]
    y2_ref[...] = y2
    st2_ref[...] = _gstats(y2)


def _flow_s3_body(y2_ref, st2_ref, g2_ref, be2_ref, o_ref):
    h2 = _gn_apply(y2_ref[...], st2_ref[...], g2_ref[...], be2_ref[...])
    o_ref[...] = jnp.max(h2.reshape(FBLK, NSAMPLE, 64), axis=1)


def _flow_stage(r1, px1, b1, g1, be1, w2, b2, g2, be2):
    grid = (B, FNB)
    rblk = pl.BlockSpec((FBLK * NSAMPLE, 128), lambda b, i: (b * FNB + i, 0))
    pblk = pl.BlockSpec((FBLK, 64), lambda b, i: (b * FNB + i, 0))
    vec = pl.BlockSpec((64,), lambda b, i: (0,))
    st_out = pl.BlockSpec((1, 1, 1, 8), lambda b, i: (b, i, 0, 0))
    st_in = pl.BlockSpec((1, FNB, 1, 8), lambda b, i: (b, 0, 0, 0))
    st_shape = jax.ShapeDtypeStruct((B, FNB, 1, 8), jnp.float32)

    st1 = pl.pallas_call(
        _flow_s1_body, grid=grid,
        in_specs=[rblk, pblk, vec],
        out_specs=st_out, out_shape=st_shape,
        compiler_params=pltpu.CompilerParams(
            dimension_semantics=("parallel", "parallel")),
    )(r1, px1, b1)

    y2, st2 = pl.pallas_call(
        _flow_s2_body, grid=grid,
        in_specs=[rblk, pblk, vec, st_in, vec, vec,
                  pl.BlockSpec((64, 64), lambda b, i: (0, 0)), vec],
        out_specs=[pl.BlockSpec((FBLK * NSAMPLE, 64),
                                lambda b, i: (b * FNB + i, 0)), st_out],
        out_shape=[jax.ShapeDtypeStruct((B * N * NSAMPLE, 64), jnp.float32),
                   st_shape],
        compiler_params=pltpu.CompilerParams(
            dimension_semantics=("parallel", "parallel")),
    )(r1, px1, b1, st1, g1, be1, w2, b2)

    return pl.pallas_call(
        _flow_s3_body, grid=grid,
        in_specs=[pl.BlockSpec((FBLK * NSAMPLE, 64),
                               lambda b, i: (b * FNB + i, 0)),
                  st_in, vec, vec],
        out_specs=pblk,
        out_shape=jax.ShapeDtypeStruct((B * N, 64), jnp.float32),
        compiler_params=pltpu.CompilerParams(
            dimension_semantics=("parallel", "parallel")),
    )(y2, st2, g2, be2)


def _a1_body(fl_ref, p0_ref, w1_ref, a1_ref, px_ref):
    w1 = w1_ref[...]
    a1 = (jnp.dot(fl_ref[...], w1[:, 0:3].T, preferred_element_type=jnp.float32)
          + jnp.dot(p0_ref[...], w1[:, 3:6].T, preferred_element_type=jnp.float32))
    a1_ref[...] = jnp.concatenate(
        [a1, jnp.zeros((a1.shape[0], 64), jnp.float32)], axis=1)
    px_ref[...] = jnp.dot(p0_ref[...], w1[:, 3:6].T,
                          preferred_element_type=jnp.float32)


def _a1_pre(fl0, p0s, w1):
    blk = lambda d: pl.BlockSpec((N, d), lambda b: (b, 0))
    return pl.pallas_call(
        _a1_body,
        grid=(B,),
        in_specs=[blk(3), blk(3), pl.BlockSpec((64, 6), lambda b: (0, 0))],
        out_specs=[blk(128), blk(64)],
        out_shape=[jax.ShapeDtypeStruct((B * N, 128), jnp.float32),
                   jax.ShapeDtypeStruct((B * N, 64), jnp.float32)],
        compiler_params=pltpu.CompilerParams(
            dimension_semantics=("parallel",)),
    )(fl0, p0s, w1)


def _gru_pre_body(c_ref, f_ref, s_ref, p_ref, wz_ref, wr_ref, azr_ref, px_ref):
    hs = jnp.concatenate([c_ref[...], f_ref[...], s_ref[...]], axis=1)
    p0 = p_ref[...]
    wz = wz_ref[...]
    wr = wr_ref[...]
    az = (jnp.dot(hs, wz[:, 0:192].T, preferred_element_type=jnp.float32)
          + jnp.dot(p0, wz[:, 192:195].T, preferred_element_type=jnp.float32))
    ar = (jnp.dot(hs, wr[:, 0:192].T, preferred_element_type=jnp.float32)
          + jnp.dot(p0, wr[:, 192:195].T, preferred_element_type=jnp.float32))
    azr_ref[...] = jnp.concatenate([az, ar], axis=1)
    pxz = jnp.dot(p0, wz[:, 192:195].T, preferred_element_type=jnp.float32)
    pxr = jnp.dot(p0, wr[:, 192:195].T, preferred_element_type=jnp.float32)
    px_ref[...] = jnp.concatenate([pxz, pxr], axis=1)


def _gru_pre(c, f, s, p, wz, wr):
    blk = lambda d: pl.BlockSpec((N, d), lambda b: (b, 0))
    return pl.pallas_call(
        _gru_pre_body,
        grid=(B,),
        in_specs=[blk(64), blk(64), blk(64), blk(3),
                  pl.BlockSpec((64, 195), lambda b: (0, 0)),
                  pl.BlockSpec((64, 195), lambda b: (0, 0))],
        out_specs=[blk(128), blk(128)],
        out_shape=[jax.ShapeDtypeStruct((B * N, 128), jnp.float32),
                   jax.ShapeDtypeStruct((B * N, 128), jnp.float32)],
        compiler_params=pltpu.CompilerParams(
            dimension_semantics=("parallel",)),
    )(c, f, s, p, wz, wr)


ZBLK = 1024


def _gru_mid_body(g_ref, px_ref, c_ref, f_ref, s_ref, p_ref,
                  wq_ref, bz_ref, br_ref, sq_ref, z_ref, pxq_ref):
    mzr = jnp.max(g_ref[...].reshape(ZBLK, NSAMPLE, 128), axis=1)
    px = px_ref[...]
    z = jax.nn.sigmoid(mzr[:, 0:64] - px[:, 0:64] + bz_ref[...])
    r = jax.nn.sigmoid(mzr[:, 64:128] - px[:, 64:128] + br_ref[...])
    st = s_ref[...]
    rs = r * st
    feat = jnp.concatenate([c_ref[...], f_ref[...]], axis=1)
    wq = wq_ref[...]
    p0 = p_ref[...]
    sq = (jnp.dot(feat, wq[:, 0:128].T, preferred_element_type=jnp.float32)
          + jnp.dot(rs, wq[:, 128:192].T, preferred_element_type=jnp.float32)
          + jnp.dot(p0, wq[:, 192:195].T, preferred_element_type=jnp.float32))
    sq_ref[...] = jnp.concatenate(
        [sq, jnp.zeros((sq.shape[0], 64), jnp.float32)], axis=1)
    z_ref[...] = z
    pxq_ref[...] = jnp.dot(p0, wq[:, 192:195].T, preferred_element_type=jnp.float32)


def _gru_mid(gzr, px, c, f, s, p, wq, bz, br):
    nb = (B * N) // ZBLK
    blk = lambda d: pl.BlockSpec((ZBLK, d), lambda g: (g, 0))
    vec = pl.BlockSpec((64,), lambda g: (0,))
    return pl.pallas_call(
        _gru_mid_body,
        grid=(nb,),
        in_specs=[pl.BlockSpec((ZBLK * NSAMPLE, 128), lambda g: (g, 0)),
                  blk(128), blk(64), blk(64), blk(64), blk(3),
                  pl.BlockSpec((64, 195), lambda g: (0, 0)), vec, vec],
        out_specs=[blk(128), blk(64), blk(64)],
        out_shape=[jax.ShapeDtypeStruct((B * N, 128), jnp.float32),
                   jax.ShapeDtypeStruct((B * N, 64), jnp.float32),
                   jax.ShapeDtypeStruct((B * N, 64), jnp.float32)],
        compiler_params=pltpu.CompilerParams(
            dimension_semantics=("parallel",)),
    )(gzr, px, c, f, s, p, wq, bz, br)


def _gru_fin_body(g_ref, pxq_ref, z_ref, s_ref, p0_ref, p1_ref,
                  f0_ref, f1_ref, bq_ref, wa_ref, ba_ref, gv_ref, qq_ref):
    mq = jnp.max(g_ref[...][:, 0:64].reshape(ZBLK, NSAMPLE, 64), axis=1)
    q = jnp.tanh(mq - pxq_ref[...] + bq_ref[...])
    z = z_ref[...]
    ns = (1.0 - z) * s_ref[...] + z * q
    wa = wa_ref[...]
    g_t = (jnp.dot(f0_ref[...], wa[:, 0:64].T, preferred_element_type=jnp.float32)
           + jnp.dot(p0_ref[...], wa[:, 128:131].T, preferred_element_type=jnp.float32))
    qq = (jnp.dot(f1_ref[...], wa[:, 64:128].T, preferred_element_type=jnp.float32)
          - jnp.dot(p1_ref[...], wa[:, 128:131].T, preferred_element_type=jnp.float32)
          + ba_ref[...])
    gv_ref[...] = jnp.concatenate([g_t, ns], axis=1)
    qq_ref[...] = qq


def _gru_fin(gq, pxq, z, s, p0, p1, f0, f1, bq, wa, ba):
    nb = (B * N) // ZBLK
    blk = lambda d: pl.BlockSpec((ZBLK, d), lambda g: (g, 0))
    vec = pl.BlockSpec((64,), lambda g: (0,))
    return pl.pallas_call(
        _gru_fin_body,
        grid=(nb,),
        in_specs=[pl.BlockSpec((ZBLK * NSAMPLE, 128), lambda g: (g, 0)),
                  blk(64), blk(64), blk(64), blk(3), blk(3), blk(64), blk(64),
                  vec, pl.BlockSpec((64, 131), lambda g: (0, 0)), vec],
        out_specs=[blk(128), blk(64)],
        out_shape=[jax.ShapeDtypeStruct((B * N, 128), jnp.float32),
                   jax.ShapeDtypeStruct((B * N, 64), jnp.float32)],
        compiler_params=pltpu.CompilerParams(
            dimension_semantics=("parallel",)),
    )(gq, pxq, z, s, p0, p1, f0, f1, bq, wa, ba)


ABLK = 512


def _align_body(r2_ref, qq_ref, w2_ref, b2_ref, o_ref):
    r2 = r2_ref[...]
    qq = qq_ref[...]
    qb = jnp.broadcast_to(qq[:, None, :], (ABLK, NSAMPLE, 64)).reshape(
        ABLK * NSAMPLE, 64)
    h = _leaky(r2[:, 0:64] + qb)
    y = jnp.dot(h, w2_ref[...].T, preferred_element_type=jnp.float32) + b2_ref[...]
    y3 = y.reshape(ABLK, NSAMPLE, 64)
    m = jnp.max(y3, axis=1, keepdims=True)
    e = jnp.exp(y3 - m)
    w = e / jnp.sum(e, axis=1, keepdims=True)
    v3 = r2[:, 64:128].reshape(ABLK, NSAMPLE, 64)
    o_ref[...] = jnp.sum(w * v3, axis=1)


def _align(r2, qq, w2, b2):
    nb = (B * N) // ABLK
    return pl.pallas_call(
        _align_body,
        grid=(nb,),
        in_specs=[pl.BlockSpec((ABLK * NSAMPLE, 128), lambda g: (g, 0)),
                  pl.BlockSpec((ABLK, 64), lambda g: (g, 0)),
                  pl.BlockSpec((64, 64), lambda g: (0, 0)),
                  pl.BlockSpec((64,), lambda g: (0,))],
        out_specs=pl.BlockSpec((ABLK, 64), lambda g: (g, 0)),
        out_shape=jax.ShapeDtypeStruct((B * N, 64), jnp.float32),
        compiler_params=pltpu.CompilerParams(
            dimension_semantics=("parallel",)),
    )(r2, qq, w2, b2)


def kernel(xyz0, xyz1, state, corr0, feat0, feat1, flow0,
           flow_w1, flow_b1, flow_g1, flow_be1,
           flow_w2, flow_b2, flow_g2, flow_be2,
           convz_w, convz_b, convr_w, convr_b, convq_w, convq_b,
           interp_w1, interp_b1, interp_w2, interp_b2):
    t = lambda x: x.transpose(0, 2, 1).reshape(B * N, -1)
    p0s = t(xyz0)
    p1s = t(xyz1)
    sts = t(state)
    c0s = t(corr0)
    f0s = t(feat0)
    f1s = t(feat1)
    fl0 = t(flow0)

    idx0 = jax.vmap(_knn)(p0s.reshape(B, N, 3), p0s.reshape(B, N, 3))
    idx1 = jax.vmap(_knn)(p1s.reshape(B, N, 3), p0s.reshape(B, N, 3))
    offs = (jnp.arange(B, dtype=jnp.int32) * N)[:, None, None]
    idxg0 = (idx0 + offs).reshape(-1)
    idxg1 = (idx1 + offs).reshape(-1)

    a1, px1 = _a1_pre(fl0, p0s, flow_w1)
    r1 = _sc_gather(a1, idxg0)
    ff0 = _flow_stage(r1, px1, flow_b1, flow_g1, flow_be1,
                      flow_w2, flow_b2, flow_g2, flow_be2)

    azr, px = _gru_pre(c0s, ff0, sts, p0s, convz_w, convr_w)
    gzr = _sc_gather(azr, idxg0)
    sq, z, pxq = _gru_mid(gzr, px, c0s, ff0, sts, p0s, convq_w,
                          convz_b, convr_b)
    gq = _sc_gather(sq, idxg0)
    gv, qq = _gru_fin(gq, pxq, z, sts, p0s, p1s, f0s, f1s,
                      convq_b, interp_w1, interp_b1)

    r2 = _sc_gather(gv, idxg1)
    out = _align(r2, qq, interp_w2, interp_b2)
    return out.reshape(B, N, 64).transpose(0, 2, 1)


# speedup vs baseline: 17.2486x; 1.0779x over previous
import functools
import jax
import jax.numpy as jnp
from jax import lax
from jax.experimental import pallas as pl
from jax.experimental.pallas import tpu as pltpu
from jax.experimental.pallas import tpu_sc as plsc

NSAMPLE = 16
HID = 64
N = 4096
B = 2
QBLK = 256
EPS = 1e-5

_PAR1 = pltpu.CompilerParams(dimension_semantics=("parallel",))


def _leaky(x):
    return jnp.where(x >= 0, x, 0.1 * x)


def _knn_body(q_ref, rt_ref, o_ref):
    q = q_ref[...]
    rt = rt_ref[...]
    n = rt.shape[1]
    qn = jnp.sum(q * q, axis=1, keepdims=True)
    rn = jnp.sum(rt * rt, axis=0, keepdims=True)
    d = qn + rn - 2.0 * jnp.dot(q, rt, preferred_element_type=jnp.float32)
    iota = lax.broadcasted_iota(jnp.int32, d.shape, 1)
    cols = []
    for _ in range(NSAMPLE):
        m = jnp.min(d, axis=1, keepdims=True)
        mi = jnp.where(d <= m, iota, jnp.int32(n))
        j = jnp.min(mi, axis=1, keepdims=True)
        cols.append(j)
        d = jnp.where(iota == j, jnp.float32(jnp.inf), d)
    o_ref[...] = jnp.concatenate(cols, axis=1)


def _knn(queries, refs):
    idx = pl.pallas_call(
        _knn_body,
        grid=(N // QBLK,),
        in_specs=[pl.BlockSpec((QBLK, 3), lambda i: (i, 0)),
                  pl.BlockSpec((3, N), lambda i: (0, 0))],
        out_specs=pl.BlockSpec((QBLK, NSAMPLE), lambda i: (i, 0)),
        out_shape=jax.ShapeDtypeStruct((N, NSAMPLE), jnp.int32),
        compiler_params=_PAR1,
    )(queries, refs.T)
    return idx.reshape(-1)


def _sc_gather(table, idx):
    M = idx.shape[0]
    D = table.shape[1]
    NW = 32
    per_w = M // NW
    ch = min(per_w, 512)
    mesh = plsc.VectorSubcoreMesh(core_axis_name="c", subcore_axis_name="s")

    @functools.partial(
        pl.kernel, mesh=mesh,
        out_type=jax.ShapeDtypeStruct((M, D), jnp.float32),
        scratch_types=[
            pltpu.VMEM((ch,), jnp.int32),
            pltpu.VMEM((ch, D), jnp.float32),
            pltpu.SemaphoreType.DMA,
        ],
    )
    def k(table_hbm, idx_hbm, out_hbm, idx_v, rows_v, sem):
        wid = lax.axis_index("s") * 2 + lax.axis_index("c")
        base = wid * per_w

        @pl.loop(0, per_w, step=ch)
        def _(off):
            pltpu.sync_copy(idx_hbm.at[pl.ds(base + off, ch)], idx_v)
            pltpu.async_copy(table_hbm.at[idx_v], rows_v, sem).wait()
            pltpu.sync_copy(rows_v, out_hbm.at[pl.ds(base + off, ch)])

    return k(table, idx)


FBLK = 1024
FNB = N // FBLK
_FCNT = float(N * NSAMPLE * 16)


def _y1_block(r1, px, b1):
    pb = jnp.broadcast_to(px[:, None, :], (FBLK, NSAMPLE, 64)).reshape(
        FBLK * NSAMPLE, 64)
    return r1[:, 0:64] - pb + b1


def _gstats(y):
    parts = []
    for g in range(4):
        blkg = y[:, 16 * g:16 * (g + 1)]
        parts.append(jnp.sum(blkg).reshape(1, 1, 1))
    for g in range(4):
        blkg = y[:, 16 * g:16 * (g + 1)]
        parts.append(jnp.sum(blkg * blkg).reshape(1, 1, 1))
    return jnp.concatenate(parts, axis=2)


def _gn_apply(y, stats, gamma, beta):
    s = jnp.sum(stats.reshape(FNB, 8), axis=0)
    outs = []
    for g in range(4):
        m = s[g] / _FCNT
        v = s[4 + g] / _FCNT - m * m
        blkg = y[:, 16 * g:16 * (g + 1)]
        outs.append((blkg - m) * lax.rsqrt(v + EPS))
    yn = jnp.concatenate(outs, axis=1) * gamma + beta
    return _leaky(yn)


def _flow_s1_body(r1_ref, px_ref, b1_ref, st_ref):
    st_ref[...] = _gstats(_y1_block(r1_ref[...], px_ref[...], b1_ref[...]))


def _flow_s2_body(r1_ref, px_ref, b1_ref, st1_ref, g1_ref, be1_ref,
                  w2_ref, b2_ref, y2_ref, st2_ref):
    y1 = _y1_block(r1_ref[...], px_ref[...], b1_ref[...])
    h = _gn_apply(y1, st1_ref[...], g1_ref[...], be1_ref[...])
    y2 = jnp.dot(h, w2_ref[...].T, preferred_element_type=jnp.float32) \
        + b2_ref[...]
    y2_ref[...] = y2
    st2_ref[...] = _gstats(y2)


def _flow_s3_body(y2_ref, st2_ref, g2_ref, be2_ref, c_ref, s_ref, p_ref,
                  wz_ref, wr_ref, o_ref, azr_ref, px_ref):
    h2 = _gn_apply(y2_ref[...], st2_ref[...], g2_ref[...], be2_ref[...])
    ff = jnp.max(h2.reshape(FBLK, NSAMPLE, 64), axis=1)
    o_ref[...] = ff
    hs = jnp.concatenate([c_ref[...], ff, s_ref[...]], axis=1)
    p0 = p_ref[...]
    wz = wz_ref[...]
    wr = wr_ref[...]
    az = (jnp.dot(hs, wz[:, 0:192].T, preferred_element_type=jnp.float32)
          + jnp.dot(p0, wz[:, 192:195].T, preferred_element_type=jnp.float32))
    ar = (jnp.dot(hs, wr[:, 0:192].T, preferred_element_type=jnp.float32)
          + jnp.dot(p0, wr[:, 192:195].T, preferred_element_type=jnp.float32))
    azr_ref[...] = jnp.concatenate([az, ar], axis=1)
    pxz = jnp.dot(p0, wz[:, 192:195].T, preferred_element_type=jnp.float32)
    pxr = jnp.dot(p0, wr[:, 192:195].T, preferred_element_type=jnp.float32)
    px_ref[...] = jnp.concatenate([pxz, pxr], axis=1)


def _flow_stage(r1, px1, b1, g1, be1, w2, b2, g2, be2, c, s, p, wz, wr):
    grid = (FNB,)
    rblk = pl.BlockSpec((FBLK * NSAMPLE, 128), lambda i: (i, 0))
    yblk = pl.BlockSpec((FBLK * NSAMPLE, 64), lambda i: (i, 0))
    pblk = pl.BlockSpec((FBLK, 64), lambda i: (i, 0))
    vec = pl.BlockSpec((64,), lambda i: (0,))
    st_out = pl.BlockSpec((1, 1, 8), lambda i: (i, 0, 0))
    st_in = pl.BlockSpec((FNB, 1, 8), lambda i: (0, 0, 0))
    st_shape = jax.ShapeDtypeStruct((FNB, 1, 8), jnp.float32)

    st1 = pl.pallas_call(
        _flow_s1_body, grid=grid,
        in_specs=[rblk, pblk, vec],
        out_specs=st_out, out_shape=st_shape,
        compiler_params=_PAR1,
    )(r1, px1, b1)

    y2, st2 = pl.pallas_call(
        _flow_s2_body, grid=grid,
        in_specs=[rblk, pblk, vec, st_in, vec, vec,
                  pl.BlockSpec((64, 64), lambda i: (0, 0)), vec],
        out_specs=[yblk, st_out],
        out_shape=[jax.ShapeDtypeStruct((N * NSAMPLE, 64), jnp.float32),
                   st_shape],
        compiler_params=_PAR1,
    )(r1, px1, b1, st1, g1, be1, w2, b2)

    return pl.pallas_call(
        _flow_s3_body, grid=grid,
        in_specs=[yblk, st_in, vec, vec, pblk, pblk,
                  pl.BlockSpec((FBLK, 3), lambda i: (i, 0)),
                  pl.BlockSpec((64, 195), lambda i: (0, 0)),
                  pl.BlockSpec((64, 195), lambda i: (0, 0))],
        out_specs=[pblk, pl.BlockSpec((FBLK, 128), lambda i: (i, 0)),
                   pl.BlockSpec((FBLK, 128), lambda i: (i, 0))],
        out_shape=[jax.ShapeDtypeStruct((N, 64), jnp.float32),
                   jax.ShapeDtypeStruct((N, 128), jnp.float32),
                   jax.ShapeDtypeStruct((N, 128), jnp.float32)],
        compiler_params=_PAR1,
    )(y2, st2, g2, be2, c, s, p, wz, wr)


def _a1_body(fl_ref, p0_ref, w1_ref, a1_ref, px_ref):
    w1 = w1_ref[...]
    a1 = (jnp.dot(fl_ref[...], w1[:, 0:3].T, preferred_element_type=jnp.float32)
          + jnp.dot(p0_ref[...], w1[:, 3:6].T, preferred_element_type=jnp.float32))
    a1_ref[...] = jnp.concatenate(
        [a1, jnp.zeros((a1.shape[0], 64), jnp.float32)], axis=1)
    px_ref[...] = jnp.dot(p0_ref[...], w1[:, 3:6].T,
                          preferred_element_type=jnp.float32)


def _a1_pre(fl0, p0, w1):
    blk = lambda d: pl.BlockSpec((N, d), lambda: (0, 0))
    return pl.pallas_call(
        _a1_body,
        in_specs=[blk(3), blk(3), pl.BlockSpec((64, 6), lambda: (0, 0))],
        out_specs=[blk(128), blk(64)],
        out_shape=[jax.ShapeDtypeStruct((N, 128), jnp.float32),
                   jax.ShapeDtypeStruct((N, 64), jnp.float32)],
    )(fl0, p0, w1)


ZBLK = 1024
ZNB = N // ZBLK


def _gru_mid_body(g_ref, px_ref, c_ref, f_ref, s_ref, p_ref,
                  wq_ref, bz_ref, br_ref, sq_ref, z_ref, pxq_ref):
    mzr = jnp.max(g_ref[...].reshape(ZBLK, NSAMPLE, 128), axis=1)
    px = px_ref[...]
    z = jax.nn.sigmoid(mzr[:, 0:64] - px[:, 0:64] + bz_ref[...])
    r = jax.nn.sigmoid(mzr[:, 64:128] - px[:, 64:128] + br_ref[...])
    st = s_ref[...]
    rs = r * st
    feat = jnp.concatenate([c_ref[...], f_ref[...]], axis=1)
    wq = wq_ref[...]
    p0 = p_ref[...]
    sq = (jnp.dot(feat, wq[:, 0:128].T, preferred_element_type=jnp.float32)
          + jnp.dot(rs, wq[:, 128:192].T, preferred_element_type=jnp.float32)
          + jnp.dot(p0, wq[:, 192:195].T, preferred_element_type=jnp.float32))
    sq_ref[...] = jnp.concatenate(
        [sq, jnp.zeros((sq.shape[0], 64), jnp.float32)], axis=1)
    z_ref[...] = z
    pxq_ref[...] = jnp.dot(p0, wq[:, 192:195].T,
                           preferred_element_type=jnp.float32)


def _gru_mid(gzr, px, c, f, s, p, wq, bz, br):
    blk = lambda d: pl.BlockSpec((ZBLK, d), lambda g: (g, 0))
    vec = pl.BlockSpec((64,), lambda g: (0,))
    return pl.pallas_call(
        _gru_mid_body,
        grid=(ZNB,),
        in_specs=[pl.BlockSpec((ZBLK * NSAMPLE, 128), lambda g: (g, 0)),
                  blk(128), blk(64), blk(64), blk(64), blk(3),
                  pl.BlockSpec((64, 195), lambda g: (0, 0)), vec, vec],
        out_specs=[blk(128), blk(64), blk(64)],
        out_shape=[jax.ShapeDtypeStruct((N, 128), jnp.float32),
                   jax.ShapeDtypeStruct((N, 64), jnp.float32),
                   jax.ShapeDtypeStruct((N, 64), jnp.float32)],
        compiler_params=_PAR1,
    )(gzr, px, c, f, s, p, wq, bz, br)


def _gru_fin_body(g_ref, pxq_ref, z_ref, s_ref, p0_ref, p1_ref,
                  f0_ref, f1_ref, bq_ref, wa_ref, ba_ref, gv_ref, qq_ref):
    mq = jnp.max(g_ref[...][:, 0:64].reshape(ZBLK, NSAMPLE, 64), axis=1)
    q = jnp.tanh(mq - pxq_ref[...] + bq_ref[...])
    z = z_ref[...]
    ns = (1.0 - z) * s_ref[...] + z * q
    wa = wa_ref[...]
    g_t = (jnp.dot(f0_ref[...], wa[:, 0:64].T, preferred_element_type=jnp.float32)
           + jnp.dot(p0_ref[...], wa[:, 128:131].T, preferred_element_type=jnp.float32))
    qq = (jnp.dot(f1_ref[...], wa[:, 64:128].T, preferred_element_type=jnp.float32)
          - jnp.dot(p1_ref[...], wa[:, 128:131].T, preferred_element_type=jnp.float32)
          + ba_ref[...])
    gv_ref[...] = jnp.concatenate([g_t, ns], axis=1)
    qq_ref[...] = qq


def _gru_fin(gq, pxq, z, s, p0, p1, f0, f1, bq, wa, ba):
    blk = lambda d: pl.BlockSpec((ZBLK, d), lambda g: (g, 0))
    vec = pl.BlockSpec((64,), lambda g: (0,))
    return pl.pallas_call(
        _gru_fin_body,
        grid=(ZNB,),
        in_specs=[pl.BlockSpec((ZBLK * NSAMPLE, 128), lambda g: (g, 0)),
                  blk(64), blk(64), blk(64), blk(3), blk(3), blk(64), blk(64),
                  vec, pl.BlockSpec((64, 131), lambda g: (0, 0)), vec],
        out_specs=[blk(128), blk(64)],
        out_shape=[jax.ShapeDtypeStruct((N, 128), jnp.float32),
                   jax.ShapeDtypeStruct((N, 64), jnp.float32)],
        compiler_params=_PAR1,
    )(gq, pxq, z, s, p0, p1, f0, f1, bq, wa, ba)


ABLK = 512


def _align_body(r2_ref, qq_ref, w2_ref, b2_ref, o_ref):
    r2 = r2_ref[...]
    qq = qq_ref[...]
    qb = jnp.broadcast_to(qq[:, None, :], (ABLK, NSAMPLE, 64)).reshape(
        ABLK * NSAMPLE, 64)
    h = _leaky(r2[:, 0:64] + qb)
    y = jnp.dot(h, w2_ref[...].T, preferred_element_type=jnp.float32) + b2_ref[...]
    y3 = y.reshape(ABLK, NSAMPLE, 64)
    m = jnp.max(y3, axis=1, keepdims=True)
    e = jnp.exp(y3 - m)
    w = e / jnp.sum(e, axis=1, keepdims=True)
    v3 = r2[:, 64:128].reshape(ABLK, NSAMPLE, 64)
    o_ref[...] = jnp.sum(w * v3, axis=1)


def _align(r2, qq, w2, b2):
    return pl.pallas_call(
        _align_body,
        grid=(N // ABLK,),
        in_specs=[pl.BlockSpec((ABLK * NSAMPLE, 128), lambda g: (g, 0)),
                  pl.BlockSpec((ABLK, 64), lambda g: (g, 0)),
                  pl.BlockSpec((64, 64), lambda g: (0, 0)),
                  pl.BlockSpec((64,), lambda g: (0,))],
        out_specs=pl.BlockSpec((ABLK, 64), lambda g: (g, 0)),
        out_shape=jax.ShapeDtypeStruct((N, 64), jnp.float32),
        compiler_params=_PAR1,
    )(r2, qq, w2, b2)


def kernel(xyz0, xyz1, state, corr0, feat0, feat1, flow0,
           flow_w1, flow_b1, flow_g1, flow_be1,
           flow_w2, flow_b2, flow_g2, flow_be2,
           convz_w, convz_b, convr_w, convr_b, convq_w, convq_b,
           interp_w1, interp_b1, interp_w2, interp_b2):
    t = lambda x: x.transpose(0, 2, 1)
    p0a = t(xyz0)
    p1a = t(xyz1)
    sta = t(state)
    c0a = t(corr0)
    f0a = t(feat0)
    f1a = t(feat1)
    fla = t(flow0)

    outs = []
    for b in range(B):
        p0, p1, st, c0 = p0a[b], p1a[b], sta[b], c0a[b]
        f0, f1, fl = f0a[b], f1a[b], fla[b]

        idxg0 = _knn(p0, p0)
        idxg1 = _knn(p1, p0)

        a1, px1 = _a1_pre(fl, p0, flow_w1)
        r1 = _sc_gather(a1, idxg0)
        ff0, azr, px = _flow_stage(
            r1, px1, flow_b1, flow_g1, flow_be1,
            flow_w2, flow_b2, flow_g2, flow_be2,
            c0, st, p0, convz_w, convr_w)

        gzr = _sc_gather(azr, idxg0)
        sq, z, pxq = _gru_mid(gzr, px, c0, ff0, st, p0, convq_w,
                              convz_b, convr_b)
        gq = _sc_gather(sq, idxg0)
        gv, qq = _gru_fin(gq, pxq, z, st, p0, p1, f0, f1,
                          convq_b, interp_w1, interp_b1)

        r2 = _sc_gather(gv, idxg1)
        outs.append(_align(r2, qq, interp_w2, interp_b2))

    return jnp.stack(outs).transpose(0, 2, 1)
